# baseline (speedup 1.0000x reference)
import functools
import jax
import jax.numpy as jnp
from jax import lax
from jax.experimental import pallas as pl
from jax.experimental.pallas import tpu as pltpu

_BS = 8
_NSTK = 32
_NPNT = 64
_ND = _NSTK * _NPNT
_CU = 128
_CO = 128
_KDN = 10
_KSP = 2
_TR = 256
_NT = _ND // _TR
_SLOPE = 0.2
_BIG = 1e30


def _leaky(x):
    return jnp.where(x >= 0, x, _SLOPE * x)


def _split_bf16(x):
    hi = x.astype(jnp.bfloat16)
    lo = (x - hi.astype(jnp.float32)).astype(jnp.bfloat16)
    return hi, lo


def _topk_max_B(D, col, bhi, k):
    n = D.shape[1]
    co = bhi.shape[1] // 2
    M = None
    for _ in range(k):
        m = jnp.min(D, axis=1, keepdims=True)
        eq = D == m
        eqf = eq.astype(jnp.float32)
        cnt = jnp.sum(eqf, axis=1, keepdims=True)
        multi = jnp.any(cnt > 1.5)

        def _slow(_):
            jc = jnp.where(eq, col, n)
            jmin = jnp.min(jc, axis=1, keepdims=True)
            return (col == jmin).astype(jnp.float32)

        ohf = lax.cond(multi, _slow, lambda _: eqf, None)
        ohb = ohf.astype(jnp.bfloat16)
        g2 = jnp.dot(ohb, bhi, preferred_element_type=jnp.float32)
        g = g2[:, :co] + g2[:, co:]
        M = g if M is None else jnp.maximum(M, g)
        D = D + ohf * _BIG
    return M


def _dense_body(xt_ref, w1_ref, w2_ref, bdn_ref, wc_ref, bc_ref,
                out_ref, b_scr, bhl_scr, sqc_scr):
    t = pl.program_id(1)
    xt = xt_ref[0]

    @pl.when(t == 0)
    def _():
        B = jnp.dot(xt, w2_ref[...],
                    preferred_element_type=jnp.float32, precision=lax.Precision.HIGHEST)
        b_scr[...] = B
        bhi, blo = _split_bf16(B)
        bhl_scr[...] = jnp.concatenate([bhi, blo], axis=1)
        xsq = xt * xt
        sqc_scr[...] = lax.dot_general(
            jnp.ones((1, _CU), jnp.float32), xsq,
            (((1,), (1,)), ((), ())),
            preferred_element_type=jnp.float32, precision=lax.Precision.HIGHEST)

    x_rows = xt_ref[0, pl.ds(t * _TR, _TR), :]
    inner = lax.dot_general(x_rows, xt, (((1,), (1,)), ((), ())),
                            preferred_element_type=jnp.float32)
    sqr = jnp.sum(x_rows * x_rows, axis=1, keepdims=True)
    D = sqr + sqc_scr[...] - 2.0 * inner

    col = lax.broadcasted_iota(jnp.int32, (_TR, _ND), 1)
    M = _topk_max_B(D, col, bhl_scr[...], _KDN)

    A = jnp.dot(x_rows, w1_ref[...], preferred_element_type=jnp.float32, precision=lax.Precision.HIGHEST)
    Brows = b_scr[pl.ds(t * _TR, _TR), :]
    h = _leaky(A - Brows + bdn_ref[...] + M)

    r2 = (_TR // _NPNT) * (_NPNT // 2)
    ri = lax.broadcasted_iota(jnp.int32, (r2, _TR), 0)
    ci = lax.broadcasted_iota(jnp.int32, (r2, _TR), 1)
    s = ri // 32
    p = ri % 32
    base = s * _NPNT + 2 * p
    sel_e = (ci == base).astype(jnp.bfloat16)
    sel_o = (ci == base + 1).astype(jnp.bfloat16)
    sel_m = ((ci == base - 1) & (p != 0)).astype(jnp.bfloat16)
    hhi, hlo = _split_bf16(h)

    def _sel(s):
        return (jnp.dot(s, hhi, preferred_element_type=jnp.float32)
                + jnp.dot(s, hlo, preferred_element_type=jnp.float32))

    E = _sel(sel_e)
    O = _sel(sel_o)
    Om = _sel(sel_m)
    out = (jnp.dot(Om, wc_ref[0], preferred_element_type=jnp.float32, precision=lax.Precision.HIGHEST)
           + jnp.dot(E, wc_ref[1], preferred_element_type=jnp.float32, precision=lax.Precision.HIGHEST)
           + jnp.dot(O, wc_ref[2], preferred_element_type=jnp.float32, precision=lax.Precision.HIGHEST)
           + bc_ref[...])
    out_ref[0] = jax.nn.gelu(out)


def _sparse_body(sp_ref, dn_ref, w1_ref, w2_ref, bsp_ref, out_ref):
    pooled = jnp.max(dn_ref[0], axis=1)
    xts = jnp.concatenate([sp_ref[0], pooled], axis=1)
    B = jnp.dot(xts, w2_ref[...], preferred_element_type=jnp.float32, precision=lax.Precision.HIGHEST)
    xsq = xts * xts
    sqr = jnp.sum(xsq, axis=1, keepdims=True)
    sqc = lax.dot_general(jnp.ones((1, _CU), jnp.float32), xsq,
                          (((1,), (1,)), ((), ())),
                          preferred_element_type=jnp.float32, precision=lax.Precision.HIGHEST)
    inner = lax.dot_general(xts, xts, (((1,), (1,)), ((), ())),
                            preferred_element_type=jnp.float32)
    D = sqr + sqc - 2.0 * inner
    col = lax.broadcasted_iota(jnp.int32, (_NSTK, _NSTK), 1)
    bhi, blo = _split_bf16(B)
    M = _topk_max_B(D, col, jnp.concatenate([bhi, blo], axis=1), _KSP)
    A = jnp.dot(xts, w1_ref[...], preferred_element_type=jnp.float32, precision=lax.Precision.HIGHEST)
    out_ref[0] = _leaky(A - B + bsp_ref[...] + M)


def kernel(sparse_fea, dense_fea, W_sp, b_sp, W_dn, b_dn, W_conv, b_conv):
    bs = sparse_fea.shape[0]
    f32 = jnp.float32

    dense_t4 = jnp.transpose(dense_fea, (0, 2, 3, 1))
    sparse_t = jnp.transpose(sparse_fea, (0, 2, 1))
    sp_bcast = jnp.broadcast_to(
        sparse_t[:, :, None, :], (bs, _NSTK, _NPNT, sparse_t.shape[2]))
    xt_dn = jnp.concatenate([dense_t4, sp_bcast], axis=-1)
    xt_dn = xt_dn.reshape(bs, _ND, _CU)
    w1_dn, w2_dn = W_dn[:_CU], W_dn[_CU:]
    w1_sp, w2_sp = W_sp[:_CU], W_sp[_CU:]
    wc = jnp.transpose(W_conv[:, :, 0, :], (2, 1, 0))
    bdn2 = b_dn.reshape(1, _CO)
    bsp2 = b_sp.reshape(1, _CO)
    bc2 = b_conv.reshape(1, _CO)

    r2 = (_TR // _NPNT) * (_NPNT // 2)
    conv_out = pl.pallas_call(
        _dense_body,
        grid=(bs, _NT),
        in_specs=[
            pl.BlockSpec((1, _ND, _CU), lambda b, t: (b, 0, 0)),
            pl.BlockSpec((_CU, _CO), lambda b, t: (0, 0)),
            pl.BlockSpec((_CU, _CO), lambda b, t: (0, 0)),
            pl.BlockSpec((1, _CO), lambda b, t: (0, 0)),
            pl.BlockSpec((3, _CU, _CO), lambda b, t: (0, 0, 0)),
            pl.BlockSpec((1, _CO), lambda b, t: (0, 0)),
        ],
        out_specs=pl.BlockSpec((1, r2, _CO), lambda b, t: (b, t, 0)),
        out_shape=jax.ShapeDtypeStruct((bs, _ND // 2, _CO), f32),
        scratch_shapes=[
            pltpu.VMEM((_ND, _CO), f32),
            pltpu.VMEM((_ND, 2 * _CO), jnp.bfloat16),
            pltpu.VMEM((1, _ND), f32),
        ],
        compiler_params=pltpu.CompilerParams(
            dimension_semantics=("parallel", "arbitrary")),
    )(xt_dn, w1_dn, w2_dn, bdn2, wc, bc2)

    us_nc = pl.pallas_call(
        _sparse_body,
        grid=(bs,),
        in_specs=[
            pl.BlockSpec((1, _NSTK, 64), lambda b: (b, 0, 0)),
            pl.BlockSpec((1, _NSTK, _NPNT, 64), lambda b: (b, 0, 0, 0)),
            pl.BlockSpec((_CU, _CO), lambda b: (0, 0)),
            pl.BlockSpec((_CU, _CO), lambda b: (0, 0)),
            pl.BlockSpec((1, _CO), lambda b: (0, 0)),
        ],
        out_specs=pl.BlockSpec((1, _NSTK, _CO), lambda b: (b, 0, 0)),
        out_shape=jax.ShapeDtypeStruct((bs, _NSTK, _CO), f32),
        compiler_params=pltpu.CompilerParams(
            dimension_semantics=("parallel",)),
    )(sparse_t, dense_t4, w1_sp, w2_sp, bsp2)

    us = jnp.transpose(us_nc, (0, 2, 1))
    ud = jnp.transpose(conv_out.reshape(bs, _NSTK, _NPNT // 2, _CO),
                       (0, 3, 1, 2))
    return (us, ud)


# speedup vs baseline: 1.7033x; 1.7033x over previous
import functools
import jax
import jax.numpy as jnp
from jax import lax
from jax.experimental import pallas as pl
from jax.experimental.pallas import tpu as pltpu

_BS = 8
_NSTK = 32
_NPNT = 64
_ND = _NSTK * _NPNT
_CU = 128
_CO = 128
_KDN = 10
_KSP = 2
_TR = 256
_NT = _ND // _TR
_SLOPE = 0.2
_BIG = 1e30


def _leaky(x):
    return jnp.where(x >= 0, x, _SLOPE * x)


def _split_bf16(x):
    hi = x.astype(jnp.bfloat16)
    lo = (x - hi.astype(jnp.float32)).astype(jnp.bfloat16)
    return hi, lo


def _topk_max_B(D, col, bhi, k):
    n = D.shape[1]
    co = bhi.shape[1] // 2
    M = None
    for _ in range(k):
        m = jnp.min(D, axis=1, keepdims=True)
        eq = D == m
        jc = jnp.where(eq, col, n)
        jmin = jnp.min(jc, axis=1, keepdims=True)
        ohf = (col == jmin).astype(jnp.float32)
        ohb = ohf.astype(jnp.bfloat16)
        g2 = jnp.dot(ohb, bhi, preferred_element_type=jnp.float32)
        g = g2[:, :co] + g2[:, co:]
        M = g if M is None else jnp.maximum(M, g)
        D = D + ohf * _BIG
    return M


def _dense_body(xt_ref, w1_ref, w2_ref, bdn_ref, wc_ref, bc_ref,
                out_ref, b_scr, bhl_scr, sqc_scr):
    t = pl.program_id(1)
    xt = xt_ref[0]

    @pl.when(t == 0)
    def _():
        B = jnp.dot(xt, w2_ref[...],
                    preferred_element_type=jnp.float32, precision=lax.Precision.HIGHEST)
        b_scr[...] = B
        bhi, blo = _split_bf16(B)
        bhl_scr[...] = jnp.concatenate([bhi, blo], axis=1)
        xsq = xt * xt
        sqc_scr[...] = lax.dot_general(
            jnp.ones((1, _CU), jnp.float32), xsq,
            (((1,), (1,)), ((), ())),
            preferred_element_type=jnp.float32, precision=lax.Precision.HIGHEST)

    x_rows = xt_ref[0, pl.ds(t * _TR, _TR), :]
    inner = lax.dot_general(x_rows, xt, (((1,), (1,)), ((), ())),
                            preferred_element_type=jnp.float32)
    sqr = jnp.sum(x_rows * x_rows, axis=1, keepdims=True)
    D = sqr + sqc_scr[...] - 2.0 * inner

    col = lax.broadcasted_iota(jnp.int32, (_TR, _ND), 1)
    M = _topk_max_B(D, col, bhl_scr[...], _KDN)

    A = jnp.dot(x_rows, w1_ref[...], preferred_element_type=jnp.float32, precision=lax.Precision.HIGHEST)
    Brows = b_scr[pl.ds(t * _TR, _TR), :]
    h = _leaky(A - Brows + bdn_ref[...] + M)

    r2 = (_TR // _NPNT) * (_NPNT // 2)
    ri = lax.broadcasted_iota(jnp.int32, (r2, _TR), 0)
    ci = lax.broadcasted_iota(jnp.int32, (r2, _TR), 1)
    s = ri // 32
    p = ri % 32
    base = s * _NPNT + 2 * p
    sel_e = (ci == base).astype(jnp.bfloat16)
    sel_o = (ci == base + 1).astype(jnp.bfloat16)
    sel_m = ((ci == base - 1) & (p != 0)).astype(jnp.bfloat16)
    hhi, hlo = _split_bf16(h)

    def _sel(s):
        return (jnp.dot(s, hhi, preferred_element_type=jnp.float32)
                + jnp.dot(s, hlo, preferred_element_type=jnp.float32))

    E = _sel(sel_e)
    O = _sel(sel_o)
    Om = _sel(sel_m)
    out = (jnp.dot(Om, wc_ref[0], preferred_element_type=jnp.float32, precision=lax.Precision.HIGHEST)
           + jnp.dot(E, wc_ref[1], preferred_element_type=jnp.float32, precision=lax.Precision.HIGHEST)
           + jnp.dot(O, wc_ref[2], preferred_element_type=jnp.float32, precision=lax.Precision.HIGHEST)
           + bc_ref[...])
    out_ref[0] = jax.nn.gelu(out)


def _sparse_body(sp_ref, dn_ref, w1_ref, w2_ref, bsp_ref, out_ref):
    pooled = jnp.max(dn_ref[0], axis=1)
    xts = jnp.concatenate([sp_ref[0], pooled], axis=1)
    B = jnp.dot(xts, w2_ref[...], preferred_element_type=jnp.float32, precision=lax.Precision.HIGHEST)
    xsq = xts * xts
    sqr = jnp.sum(xsq, axis=1, keepdims=True)
    sqc = lax.dot_general(jnp.ones((1, _CU), jnp.float32), xsq,
                          (((1,), (1,)), ((), ())),
                          preferred_element_type=jnp.float32, precision=lax.Precision.HIGHEST)
    inner = lax.dot_general(xts, xts, (((1,), (1,)), ((), ())),
                            preferred_element_type=jnp.float32)
    D = sqr + sqc - 2.0 * inner
    col = lax.broadcasted_iota(jnp.int32, (_NSTK, _NSTK), 1)
    bhi, blo = _split_bf16(B)
    M = _topk_max_B(D, col, jnp.concatenate([bhi, blo], axis=1), _KSP)
    A = jnp.dot(xts, w1_ref[...], preferred_element_type=jnp.float32, precision=lax.Precision.HIGHEST)
    out_ref[0] = _leaky(A - B + bsp_ref[...] + M)


def kernel(sparse_fea, dense_fea, W_sp, b_sp, W_dn, b_dn, W_conv, b_conv):
    bs = sparse_fea.shape[0]
    f32 = jnp.float32

    dense_t4 = jnp.transpose(dense_fea, (0, 2, 3, 1))
    sparse_t = jnp.transpose(sparse_fea, (0, 2, 1))
    sp_bcast = jnp.broadcast_to(
        sparse_t[:, :, None, :], (bs, _NSTK, _NPNT, sparse_t.shape[2]))
    xt_dn = jnp.concatenate([dense_t4, sp_bcast], axis=-1)
    xt_dn = xt_dn.reshape(bs, _ND, _CU)
    w1_dn, w2_dn = W_dn[:_CU], W_dn[_CU:]
    w1_sp, w2_sp = W_sp[:_CU], W_sp[_CU:]
    wc = jnp.transpose(W_conv[:, :, 0, :], (2, 1, 0))
    bdn2 = b_dn.reshape(1, _CO)
    bsp2 = b_sp.reshape(1, _CO)
    bc2 = b_conv.reshape(1, _CO)

    r2 = (_TR // _NPNT) * (_NPNT // 2)
    conv_out = pl.pallas_call(
        _dense_body,
        grid=(bs, _NT),
        in_specs=[
            pl.BlockSpec((1, _ND, _CU), lambda b, t: (b, 0, 0)),
            pl.BlockSpec((_CU, _CO), lambda b, t: (0, 0)),
            pl.BlockSpec((_CU, _CO), lambda b, t: (0, 0)),
            pl.BlockSpec((1, _CO), lambda b, t: (0, 0)),
            pl.BlockSpec((3, _CU, _CO), lambda b, t: (0, 0, 0)),
            pl.BlockSpec((1, _CO), lambda b, t: (0, 0)),
        ],
        out_specs=pl.BlockSpec((1, r2, _CO), lambda b, t: (b, t, 0)),
        out_shape=jax.ShapeDtypeStruct((bs, _ND // 2, _CO), f32),
        scratch_shapes=[
            pltpu.VMEM((_ND, _CO), f32),
            pltpu.VMEM((_ND, 2 * _CO), jnp.bfloat16),
            pltpu.VMEM((1, _ND), f32),
        ],
        compiler_params=pltpu.CompilerParams(
            dimension_semantics=("parallel", "arbitrary")),
    )(xt_dn, w1_dn, w2_dn, bdn2, wc, bc2)

    us_nc = pl.pallas_call(
        _sparse_body,
        grid=(bs,),
        in_specs=[
            pl.BlockSpec((1, _NSTK, 64), lambda b: (b, 0, 0)),
            pl.BlockSpec((1, _NSTK, _NPNT, 64), lambda b: (b, 0, 0, 0)),
            pl.BlockSpec((_CU, _CO), lambda b: (0, 0)),
            pl.BlockSpec((_CU, _CO), lambda b: (0, 0)),
            pl.BlockSpec((1, _CO), lambda b: (0, 0)),
        ],
        out_specs=pl.BlockSpec((1, _NSTK, _CO), lambda b: (b, 0, 0)),
        out_shape=jax.ShapeDtypeStruct((bs, _NSTK, _CO), f32),
        compiler_params=pltpu.CompilerParams(
            dimension_semantics=("parallel",)),
    )(sparse_t, dense_t4, w1_sp, w2_sp, bsp2)

    us = jnp.transpose(us_nc, (0, 2, 1))
    ud = jnp.transpose(conv_out.reshape(bs, _NSTK, _NPNT // 2, _CO),
                       (0, 3, 1, 2))
    return (us, ud)


# speedup vs baseline: 1.7481x; 1.0263x over previous
import functools
import jax
import jax.numpy as jnp
from jax import lax
from jax.experimental import pallas as pl
from jax.experimental.pallas import tpu as pltpu

_BS = 8
_NSTK = 32
_NPNT = 64
_ND = _NSTK * _NPNT
_CU = 128
_CO = 128
_KDN = 10
_KSP = 2
_TR = 512
_NT = _ND // _TR
_SLOPE = 0.2
_BIG = 1e30


def _leaky(x):
    return jnp.where(x >= 0, x, _SLOPE * x)


def _split_bf16(x):
    hi = x.astype(jnp.bfloat16)
    lo = (x - hi.astype(jnp.float32)).astype(jnp.bfloat16)
    return hi, lo


def _topk_max_B(D, col, bhi, k):
    n = D.shape[1]
    co = bhi.shape[1] // 2
    M = None
    for _ in range(k):
        m = jnp.min(D, axis=1, keepdims=True)
        eq = D == m
        jc = jnp.where(eq, col, n)
        jmin = jnp.min(jc, axis=1, keepdims=True)
        ohf = (col == jmin).astype(jnp.float32)
        ohb = ohf.astype(jnp.bfloat16)
        g2 = jnp.dot(ohb, bhi, preferred_element_type=jnp.float32)
        g = g2[:, :co] + g2[:, co:]
        M = g if M is None else jnp.maximum(M, g)
        D = D + ohf * _BIG
    return M


def _dense_body(xt_ref, w1_ref, w2_ref, bdn_ref, wc_ref, bc_ref,
                out_ref, b_scr, bhl_scr, sqc_scr):
    t = pl.program_id(1)
    xt = xt_ref[0]

    @pl.when(t == 0)
    def _():
        B = jnp.dot(xt, w2_ref[...],
                    preferred_element_type=jnp.float32, precision=lax.Precision.HIGHEST)
        b_scr[...] = B
        bhi, blo = _split_bf16(B)
        bhl_scr[...] = jnp.concatenate([bhi, blo], axis=1)
        xsq = xt * xt
        sqc_scr[...] = lax.dot_general(
            jnp.ones((1, _CU), jnp.float32), xsq,
            (((1,), (1,)), ((), ())),
            preferred_element_type=jnp.float32, precision=lax.Precision.HIGHEST)

    x_rows = xt_ref[0, pl.ds(t * _TR, _TR), :]
    inner = lax.dot_general(x_rows, xt, (((1,), (1,)), ((), ())),
                            preferred_element_type=jnp.float32)
    sqr = jnp.sum(x_rows * x_rows, axis=1, keepdims=True)
    D = sqr + sqc_scr[...] - 2.0 * inner

    col = lax.broadcasted_iota(jnp.int32, (_TR, _ND), 1)
    M = _topk_max_B(D, col, bhl_scr[...], _KDN)

    A = jnp.dot(x_rows, w1_ref[...], preferred_element_type=jnp.float32, precision=lax.Precision.HIGHEST)
    Brows = b_scr[pl.ds(t * _TR, _TR), :]
    h = _leaky(A - Brows + bdn_ref[...] + M)

    r2 = (_TR // _NPNT) * (_NPNT // 2)
    ri = lax.broadcasted_iota(jnp.int32, (r2, _TR), 0)
    ci = lax.broadcasted_iota(jnp.int32, (r2, _TR), 1)
    s = ri // 32
    p = ri % 32
    base = s * _NPNT + 2 * p
    sel_e = (ci == base).astype(jnp.bfloat16)
    sel_o = (ci == base + 1).astype(jnp.bfloat16)
    sel_m = ((ci == base - 1) & (p != 0)).astype(jnp.bfloat16)
    hhi, hlo = _split_bf16(h)

    def _sel(s):
        return (jnp.dot(s, hhi, preferred_element_type=jnp.float32)
                + jnp.dot(s, hlo, preferred_element_type=jnp.float32))

    E = _sel(sel_e)
    O = _sel(sel_o)
    Om = _sel(sel_m)
    out = (jnp.dot(Om, wc_ref[0], preferred_element_type=jnp.float32, precision=lax.Precision.HIGHEST)
           + jnp.dot(E, wc_ref[1], preferred_element_type=jnp.float32, precision=lax.Precision.HIGHEST)
           + jnp.dot(O, wc_ref[2], preferred_element_type=jnp.float32, precision=lax.Precision.HIGHEST)
           + bc_ref[...])
    out_ref[0] = jax.nn.gelu(out)


def _sparse_body(sp_ref, dn_ref, w1_ref, w2_ref, bsp_ref, out_ref):
    pooled = jnp.max(dn_ref[0], axis=1)
    xts = jnp.concatenate([sp_ref[0], pooled], axis=1)
    B = jnp.dot(xts, w2_ref[...], preferred_element_type=jnp.float32, precision=lax.Precision.HIGHEST)
    xsq = xts * xts
    sqr = jnp.sum(xsq, axis=1, keepdims=True)
    sqc = lax.dot_general(jnp.ones((1, _CU), jnp.float32), xsq,
                          (((1,), (1,)), ((), ())),
                          preferred_element_type=jnp.float32, precision=lax.Precision.HIGHEST)
    inner = lax.dot_general(xts, xts, (((1,), (1,)), ((), ())),
                            preferred_element_type=jnp.float32)
    D = sqr + sqc - 2.0 * inner
    col = lax.broadcasted_iota(jnp.int32, (_NSTK, _NSTK), 1)
    bhi, blo = _split_bf16(B)
    M = _topk_max_B(D, col, jnp.concatenate([bhi, blo], axis=1), _KSP)
    A = jnp.dot(xts, w1_ref[...], preferred_element_type=jnp.float32, precision=lax.Precision.HIGHEST)
    out_ref[0] = _leaky(A - B + bsp_ref[...] + M)


def kernel(sparse_fea, dense_fea, W_sp, b_sp, W_dn, b_dn, W_conv, b_conv):
    bs = sparse_fea.shape[0]
    f32 = jnp.float32

    dense_t4 = jnp.transpose(dense_fea, (0, 2, 3, 1))
    sparse_t = jnp.transpose(sparse_fea, (0, 2, 1))
    sp_bcast = jnp.broadcast_to(
        sparse_t[:, :, None, :], (bs, _NSTK, _NPNT, sparse_t.shape[2]))
    xt_dn = jnp.concatenate([dense_t4, sp_bcast], axis=-1)
    xt_dn = xt_dn.reshape(bs, _ND, _CU)
    w1_dn, w2_dn = W_dn[:_CU], W_dn[_CU:]
    w1_sp, w2_sp = W_sp[:_CU], W_sp[_CU:]
    wc = jnp.transpose(W_conv[:, :, 0, :], (2, 1, 0))
    bdn2 = b_dn.reshape(1, _CO)
    bsp2 = b_sp.reshape(1, _CO)
    bc2 = b_conv.reshape(1, _CO)

    r2 = (_TR // _NPNT) * (_NPNT // 2)
    conv_out = pl.pallas_call(
        _dense_body,
        grid=(bs, _NT),
        in_specs=[
            pl.BlockSpec((1, _ND, _CU), lambda b, t: (b, 0, 0)),
            pl.BlockSpec((_CU, _CO), lambda b, t: (0, 0)),
            pl.BlockSpec((_CU, _CO), lambda b, t: (0, 0)),
            pl.BlockSpec((1, _CO), lambda b, t: (0, 0)),
            pl.BlockSpec((3, _CU, _CO), lambda b, t: (0, 0, 0)),
            pl.BlockSpec((1, _CO), lambda b, t: (0, 0)),
        ],
        out_specs=pl.BlockSpec((1, r2, _CO), lambda b, t: (b, t, 0)),
        out_shape=jax.ShapeDtypeStruct((bs, _ND // 2, _CO), f32),
        scratch_shapes=[
            pltpu.VMEM((_ND, _CO), f32),
            pltpu.VMEM((_ND, 2 * _CO), jnp.bfloat16),
            pltpu.VMEM((1, _ND), f32),
        ],
        compiler_params=pltpu.CompilerParams(
            dimension_semantics=("parallel", "arbitrary")),
    )(xt_dn, w1_dn, w2_dn, bdn2, wc, bc2)

    us_nc = pl.pallas_call(
        _sparse_body,
        grid=(bs,),
        in_specs=[
            pl.BlockSpec((1, _NSTK, 64), lambda b: (b, 0, 0)),
            pl.BlockSpec((1, _NSTK, _NPNT, 64), lambda b: (b, 0, 0, 0)),
            pl.BlockSpec((_CU, _CO), lambda b: (0, 0)),
            pl.BlockSpec((_CU, _CO), lambda b: (0, 0)),
            pl.BlockSpec((1, _CO), lambda b: (0, 0)),
        ],
        out_specs=pl.BlockSpec((1, _NSTK, _CO), lambda b: (b, 0, 0)),
        out_shape=jax.ShapeDtypeStruct((bs, _NSTK, _CO), f32),
        compiler_params=pltpu.CompilerParams(
            dimension_semantics=("parallel",)),
    )(sparse_t, dense_t4, w1_sp, w2_sp, bsp2)

    us = jnp.transpose(us_nc, (0, 2, 1))
    ud = jnp.transpose(conv_out.reshape(bs, _NSTK, _NPNT // 2, _CO),
                       (0, 3, 1, 2))
    return (us, ud)


# speedup vs baseline: 1.8668x; 1.0679x over previous
import functools
import jax
import jax.numpy as jnp
from jax import lax
from jax.experimental import pallas as pl
from jax.experimental.pallas import tpu as pltpu

_BS = 8
_NSTK = 32
_NPNT = 64
_ND = _NSTK * _NPNT
_CU = 128
_CO = 128
_KDN = 10
_KSP = 2
_TR = 512
_NT = _ND // _TR
_SLOPE = 0.2
_BIG = 1e30


def _leaky(x):
    return jnp.where(x >= 0, x, _SLOPE * x)


def _split_bf16(x):
    hi = x.astype(jnp.bfloat16)
    lo = (x - hi.astype(jnp.float32)).astype(jnp.bfloat16)
    return hi, lo


def _topk_max_B(D, col, bhi, k):
    n = float(D.shape[1])
    co = bhi.shape[1] // 2
    colf = col.astype(jnp.float32)
    M = None
    for _ in range(k):
        m = jnp.min(D, axis=1, keepdims=True)
        eq = D == m
        jc = jnp.where(eq, colf, n)
        jmin = jnp.min(jc, axis=1, keepdims=True)
        ohf = (colf == jmin).astype(jnp.float32)
        ohb = ohf.astype(jnp.bfloat16)
        g2 = jnp.dot(ohb, bhi, preferred_element_type=jnp.float32)
        g = g2[:, :co] + g2[:, co:]
        M = g if M is None else jnp.maximum(M, g)
        D = D + ohf * _BIG
    return M


def _dense_body(xt_ref, w1_ref, w2_ref, bdn_ref, wc_ref, bc_ref,
                out_ref, b_scr, bhl_scr, sqc_scr):
    t = pl.program_id(1)
    xt = xt_ref[0]

    @pl.when(t == 0)
    def _():
        B = jnp.dot(xt, w2_ref[...],
                    preferred_element_type=jnp.float32, precision=lax.Precision.HIGHEST)
        b_scr[...] = B
        bhi, blo = _split_bf16(B)
        bhl_scr[...] = jnp.concatenate([bhi, blo], axis=1)
        xsq = xt * xt
        sqc_scr[...] = lax.dot_general(
            jnp.ones((1, _CU), jnp.float32), xsq,
            (((1,), (1,)), ((), ())),
            preferred_element_type=jnp.float32, precision=lax.Precision.HIGHEST)

    x_rows = xt_ref[0, pl.ds(t * _TR, _TR), :]
    inner = lax.dot_general(x_rows, xt, (((1,), (1,)), ((), ())),
                            preferred_element_type=jnp.float32)
    sqr = jnp.sum(x_rows * x_rows, axis=1, keepdims=True)
    D = sqr + sqc_scr[...] - 2.0 * inner

    col = lax.broadcasted_iota(jnp.int32, (_TR, _ND), 1)
    M = _topk_max_B(D, col, bhl_scr[...], _KDN)

    A = jnp.dot(x_rows, w1_ref[...], preferred_element_type=jnp.float32, precision=lax.Precision.HIGHEST)
    Brows = b_scr[pl.ds(t * _TR, _TR), :]
    h = _leaky(A - Brows + bdn_ref[...] + M)

    r2 = (_TR // _NPNT) * (_NPNT // 2)
    ri = lax.broadcasted_iota(jnp.int32, (r2, _TR), 0)
    ci = lax.broadcasted_iota(jnp.int32, (r2, _TR), 1)
    s = ri // 32
    p = ri % 32
    base = s * _NPNT + 2 * p
    sel_e = (ci == base).astype(jnp.bfloat16)
    sel_o = (ci == base + 1).astype(jnp.bfloat16)
    sel_m = ((ci == base - 1) & (p != 0)).astype(jnp.bfloat16)
    hhi, hlo = _split_bf16(h)

    def _sel(s):
        return (jnp.dot(s, hhi, preferred_element_type=jnp.float32)
                + jnp.dot(s, hlo, preferred_element_type=jnp.float32))

    E = _sel(sel_e)
    O = _sel(sel_o)
    Om = _sel(sel_m)
    out = (jnp.dot(Om, wc_ref[0], preferred_element_type=jnp.float32, precision=lax.Precision.HIGHEST)
           + jnp.dot(E, wc_ref[1], preferred_element_type=jnp.float32, precision=lax.Precision.HIGHEST)
           + jnp.dot(O, wc_ref[2], preferred_element_type=jnp.float32, precision=lax.Precision.HIGHEST)
           + bc_ref[...])
    out_ref[0] = jax.nn.gelu(out)


def _sparse_body(sp_ref, dn_ref, w1_ref, w2_ref, bsp_ref, out_ref):
    pooled = jnp.max(dn_ref[0], axis=1)
    xts = jnp.concatenate([sp_ref[0], pooled], axis=1)
    B = jnp.dot(xts, w2_ref[...], preferred_element_type=jnp.float32, precision=lax.Precision.HIGHEST)
    xsq = xts * xts
    sqr = jnp.sum(xsq, axis=1, keepdims=True)
    sqc = lax.dot_general(jnp.ones((1, _CU), jnp.float32), xsq,
                          (((1,), (1,)), ((), ())),
                          preferred_element_type=jnp.float32, precision=lax.Precision.HIGHEST)
    inner = lax.dot_general(xts, xts, (((1,), (1,)), ((), ())),
                            preferred_element_type=jnp.float32)
    D = sqr + sqc - 2.0 * inner
    col = lax.broadcasted_iota(jnp.int32, (_NSTK, _NSTK), 1)
    bhi, blo = _split_bf16(B)
    M = _topk_max_B(D, col, jnp.concatenate([bhi, blo], axis=1), _KSP)
    A = jnp.dot(xts, w1_ref[...], preferred_element_type=jnp.float32, precision=lax.Precision.HIGHEST)
    out_ref[0] = _leaky(A - B + bsp_ref[...] + M)


def kernel(sparse_fea, dense_fea, W_sp, b_sp, W_dn, b_dn, W_conv, b_conv):
    bs = sparse_fea.shape[0]
    f32 = jnp.float32

    dense_t4 = jnp.transpose(dense_fea, (0, 2, 3, 1))
    sparse_t = jnp.transpose(sparse_fea, (0, 2, 1))
    sp_bcast = jnp.broadcast_to(
        sparse_t[:, :, None, :], (bs, _NSTK, _NPNT, sparse_t.shape[2]))
    xt_dn = jnp.concatenate([dense_t4, sp_bcast], axis=-1)
    xt_dn = xt_dn.reshape(bs, _ND, _CU)
    w1_dn, w2_dn = W_dn[:_CU], W_dn[_CU:]
    w1_sp, w2_sp = W_sp[:_CU], W_sp[_CU:]
    wc = jnp.transpose(W_conv[:, :, 0, :], (2, 1, 0))
    bdn2 = b_dn.reshape(1, _CO)
    bsp2 = b_sp.reshape(1, _CO)
    bc2 = b_conv.reshape(1, _CO)

    r2 = (_TR // _NPNT) * (_NPNT // 2)
    conv_out = pl.pallas_call(
        _dense_body,
        grid=(bs, _NT),
        in_specs=[
            pl.BlockSpec((1, _ND, _CU), lambda b, t: (b, 0, 0)),
            pl.BlockSpec((_CU, _CO), lambda b, t: (0, 0)),
            pl.BlockSpec((_CU, _CO), lambda b, t: (0, 0)),
            pl.BlockSpec((1, _CO), lambda b, t: (0, 0)),
            pl.BlockSpec((3, _CU, _CO), lambda b, t: (0, 0, 0)),
            pl.BlockSpec((1, _CO), lambda b, t: (0, 0)),
        ],
        out_specs=pl.BlockSpec((1, r2, _CO), lambda b, t: (b, t, 0)),
        out_shape=jax.ShapeDtypeStruct((bs, _ND // 2, _CO), f32),
        scratch_shapes=[
            pltpu.VMEM((_ND, _CO), f32),
            pltpu.VMEM((_ND, 2 * _CO), jnp.bfloat16),
            pltpu.VMEM((1, _ND), f32),
        ],
        compiler_params=pltpu.CompilerParams(
            dimension_semantics=("parallel", "arbitrary")),
    )(xt_dn, w1_dn, w2_dn, bdn2, wc, bc2)

    us_nc = pl.pallas_call(
        _sparse_body,
        grid=(bs,),
        in_specs=[
            pl.BlockSpec((1, _NSTK, 64), lambda b: (b, 0, 0)),
            pl.BlockSpec((1, _NSTK, _NPNT, 64), lambda b: (b, 0, 0, 0)),
            pl.BlockSpec((_CU, _CO), lambda b: (0, 0)),
            pl.BlockSpec((_CU, _CO), lambda b: (0, 0)),
            pl.BlockSpec((1, _CO), lambda b: (0, 0)),
        ],
        out_specs=pl.BlockSpec((1, _NSTK, _CO), lambda b: (b, 0, 0)),
        out_shape=jax.ShapeDtypeStruct((bs, _NSTK, _CO), f32),
        compiler_params=pltpu.CompilerParams(
            dimension_semantics=("parallel",)),
    )(sparse_t, dense_t4, w1_sp, w2_sp, bsp2)

    us = jnp.transpose(us_nc, (0, 2, 1))
    ud = jnp.transpose(conv_out.reshape(bs, _NSTK, _NPNT // 2, _CO),
                       (0, 3, 1, 2))
    return (us, ud)
